# baseline (speedup 1.0000x reference)
import jax
import jax.numpy as jnp
import numpy as np
from jax import lax
from jax.experimental import pallas as pl
from jax.experimental.pallas import tpu as pltpu
from jax.experimental.pallas import tpu_sc as plsc

_M, _N, _K = 8, 8, 8
_NUM_ELEMENTS = _M * _K + _K * _N + 2 * _M * _N
_NUM_OPS = _M * _N * (_K + 1)
_L = 16
_NCHUNK = _NUM_OPS // _L
_NSUB = 16
_AFS_PAD = _NUM_OPS + _L


def _op_tables():
    first_src, second_src, dst = [], [], []
    for i in range(_M):
        for j in range(_N):
            c_idx = _M * _K + _K * _N + i * _N + j
            d_idx = _M * _K + _K * _N + _M * _N + i * _N + j
            first_src.append(c_idx)
            second_src.append(-1)
            dst.append(d_idx)
            for k in range(_K):
                first_src.append(i * _K + k)
                second_src.append(_M * _K + k * _N + j)
                dst.append(d_idx)
    return np.concatenate([
        np.asarray(first_src, dtype=np.int32),
        np.asarray(second_src, dtype=np.int32),
        np.asarray(dst, dtype=np.int32),
    ])


def _to_key(p):
    b = lax.bitcast_convert_type(p + 0.0, jnp.int32)
    m = lax.shift_right_arithmetic(b, 31)
    return b ^ (m & jnp.int32(0x7FFFFFFF))


def _sc_body(mem_hbm, op_hbm, tbl_hbm, out_hbm,
             op_v, skeys_iv, mem_v, tbl_v,
             st_afA, st_adA, st_afB, st_adB, st_afC, st_adC,
             part_stage, afs_v, ads_v, part_v, out_stage,
             sem_op, sem_mem, sem_tbl, sem_sc, sem_f1, sem_f2, sem_f3,
             afs_sh, ads_sh, part_sh):
    wid = lax.axis_index("s")
    has3 = (wid >= 4) & (wid < 8)

    pltpu.async_copy(op_hbm, op_v, sem_op)
    pltpu.async_copy(mem_hbm, mem_v, sem_mem)
    pltpu.async_copy(tbl_hbm, tbl_v, sem_tbl)
    pltpu.make_async_copy(op_hbm, op_v, sem_op).wait()

    def prep_body(t, carry):
        sl = pl.ds(t * _L, _L)
        skeys_iv[sl] = _to_key(lax.sort(op_v[sl]))
        return carry

    lax.fori_loop(0, _NCHUNK, prep_body, 0, unroll=4)

    cA = wid
    cB = wid + _NSUB
    cC = jnp.where(has3, wid + 2 * _NSUB - 4, wid)
    kAi = _to_key(op_v[pl.ds(cA * _L, _L)])
    kBi = _to_key(op_v[pl.ds(cB * _L, _L)])
    kCi = _to_key(op_v[pl.ds(cC * _L, _L)])
    zero = jnp.zeros((_L,), jnp.int32)

    def srch3(jc, carry):
        jbase = jc * _L
        out = []
        for acc, myki, c in zip(carry, (kAi, kBi, kCi), (cA, cB, cC)):
            thr = myki + jnp.where(jc < c, 1, 0)
            pos = zero + jbase
            for sz in (8, 4, 2, 1, 1):
                g = plsc.load_gather(skeys_iv, [pos + (sz - 1)])
                pos = pos + jnp.where(g < thr, sz, 0)
            out.append(acc + jnp.where(jc == c, 0, pos - jbase))
        return tuple(out)

    accA, accB, accC = lax.fori_loop(0, _NCHUNK, srch3,
                                 (zero, zero, zero), unroll=2)

    li = lax.iota(jnp.int32, _L)

    def diag(acc, myki):
        for m in range(_L):
            thr = myki + jnp.where(m < li, 1, 0)
            acc = acc + jnp.where(myki[m] < thr, 1, 0)
        return acc

    accA = diag(accA, kAi)
    accB = diag(accB, kBi)
    accC = diag(accC, kCi)

    pltpu.make_async_copy(mem_hbm, mem_v, sem_mem).wait()
    pltpu.make_async_copy(tbl_hbm, tbl_v, sem_tbl).wait()

    def chunk_vals(c, r, intra):
        fs = tbl_v[pl.ds(c * _L, _L)]
        ss = tbl_v[pl.ds(_NUM_OPS + c * _L, _L)]
        ds = tbl_v[pl.ds(2 * _NUM_OPS + c * _L, _L)]
        has2 = ss >= 0
        ss_c = jnp.where(has2, ss, 0)
        af = plsc.load_gather(mem_v, [fs])
        asv = plsc.load_gather(mem_v, [ss_c])
        ad = plsc.load_gather(mem_v, [ds])
        idx1 = jnp.where(r >= 1, r - 1, _NUM_OPS + li)
        hop1 = jnp.where(has2, asv - af, ad - af)
        hop2 = ad - asv
        fwd1 = jnp.maximum(hop1, 0.0)
        bwd1 = jnp.maximum(-hop1, 0.0)
        fwd2 = jnp.where(has2, jnp.maximum(hop2, 0.0), 0.0)
        bwd2 = jnp.where(has2, jnp.maximum(-hop2, 0.0), 0.0)
        intra = intra + fwd1 + bwd1 * bwd1 + fwd2 + bwd2 * bwd2
        return af, ad, idx1, intra

    afA, adA, idxA, intra = chunk_vals(cA, accA, jnp.zeros((_L,), jnp.float32))
    st_afA[...] = afA
    st_adA[...] = adA
    dA1 = pltpu.async_copy(st_afA, afs_sh.at[idxA], sem_sc)
    dA2 = pltpu.async_copy(st_adA, ads_sh.at[accA], sem_sc)

    afB, adB, idxB, intra = chunk_vals(cB, accB, intra)
    st_afB[...] = afB
    st_adB[...] = adB
    dB1 = pltpu.async_copy(st_afB, afs_sh.at[idxB], sem_sc)
    dB2 = pltpu.async_copy(st_adB, ads_sh.at[accB], sem_sc)

    part_stage[...] = intra

    @pl.when(has3)
    def _():
        afC, adC, idxC, intraC = chunk_vals(cC, accC, intra)
        st_afC[...] = afC
        st_adC[...] = adC
        pltpu.sync_copy(st_afC, afs_sh.at[idxC])
        pltpu.sync_copy(st_adC, ads_sh.at[accC])
        part_stage[...] = intraC

    dA1.wait()
    dA2.wait()
    dB1.wait()
    dB2.wait()
    pltpu.sync_copy(part_stage, part_sh.at[pl.ds(wid * _L, _L)])

    plsc.subcore_barrier()

    @pl.when(wid == 0)
    def _():
        pltpu.async_copy(afs_sh, afs_v, sem_f1)
        pltpu.async_copy(ads_sh, ads_v, sem_f2)
        pltpu.async_copy(part_sh, part_v, sem_f3)
        pltpu.make_async_copy(afs_sh, afs_v, sem_f1).wait()
        pltpu.make_async_copy(ads_sh, ads_v, sem_f2).wait()
        pltpu.make_async_copy(part_sh, part_v, sem_f3).wait()

        def inter_body(c, acc):
            sl = pl.ds(c * _L, _L)
            v = afs_v[sl] - ads_v[sl]
            fwd = jnp.maximum(v, 0.0)
            bwd = jnp.maximum(-v, 0.0)
            contrib = fwd + bwd * bwd
            gl = li + c * _L
            return acc + jnp.where(gl < _NUM_OPS - 1, contrib, 0.0)

        inter = lax.fori_loop(0, _NCHUNK, inter_body,
                              jnp.zeros((_L,), jnp.float32))

        def part_body(t, acc):
            return acc + part_v[pl.ds(t * _L, _L)]

        intra_tot = lax.fori_loop(0, _NSUB, part_body,
                                  jnp.zeros((_L,), jnp.float32))

        inter_s = jnp.sum(inter)
        intra_s = jnp.sum(intra_tot)
        out_stage[...] = jnp.where(li == 0, inter_s,
                                   jnp.where(li == 1, intra_s, 0.0))
        pltpu.sync_copy(out_stage, out_hbm)


@jax.jit
def kernel(mem_params, op_params):
    tbl = _op_tables()
    mesh = plsc.VectorSubcoreMesh(core_axis_name="c", subcore_axis_name="s",
                                  num_cores=1)
    run = pl.kernel(
        _sc_body,
        out_type=jax.ShapeDtypeStruct((_L,), jnp.float32),
        mesh=mesh,
        compiler_params=pltpu.CompilerParams(needs_layout_passes=False),
        scratch_types=[
            pltpu.VMEM((_NUM_OPS,), jnp.float32),
            pltpu.VMEM((_NUM_OPS,), jnp.int32),
            pltpu.VMEM((_NUM_ELEMENTS,), jnp.float32),
            pltpu.VMEM((3 * _NUM_OPS,), jnp.int32),
            pltpu.VMEM((_L,), jnp.float32),
            pltpu.VMEM((_L,), jnp.float32),
            pltpu.VMEM((_L,), jnp.float32),
            pltpu.VMEM((_L,), jnp.float32),
            pltpu.VMEM((_L,), jnp.float32),
            pltpu.VMEM((_L,), jnp.float32),
            pltpu.VMEM((_L,), jnp.float32),
            pltpu.VMEM((_AFS_PAD,), jnp.float32),
            pltpu.VMEM((_NUM_OPS,), jnp.float32),
            pltpu.VMEM((_NSUB * _L,), jnp.float32),
            pltpu.VMEM((_L,), jnp.float32),
            pltpu.SemaphoreType.DMA,
            pltpu.SemaphoreType.DMA,
            pltpu.SemaphoreType.DMA,
            pltpu.SemaphoreType.DMA,
            pltpu.SemaphoreType.DMA,
            pltpu.SemaphoreType.DMA,
            pltpu.SemaphoreType.DMA,
            pltpu.VMEM_SHARED((_AFS_PAD,), jnp.float32),
            pltpu.VMEM_SHARED((_NUM_OPS,), jnp.float32),
            pltpu.VMEM_SHARED((_NSUB * _L,), jnp.float32),
        ],
    )
    out = run(mem_params, op_params, jnp.asarray(tbl))
    return (out[0], out[1])


# speedup vs baseline: 1.0249x; 1.0249x over previous
import jax
import jax.numpy as jnp
import numpy as np
from jax import lax
from jax.experimental import pallas as pl
from jax.experimental.pallas import tpu as pltpu
from jax.experimental.pallas import tpu_sc as plsc

_M, _N, _K = 8, 8, 8
_NUM_ELEMENTS = _M * _K + _K * _N + 2 * _M * _N
_NUM_OPS = _M * _N * (_K + 1)
_L = 16
_NCHUNK = _NUM_OPS // _L
_NSUB = 16
_AFS_PAD = _NUM_OPS + _L


def _op_tables():
    first_src, second_src, dst = [], [], []
    for i in range(_M):
        for j in range(_N):
            c_idx = _M * _K + _K * _N + i * _N + j
            d_idx = _M * _K + _K * _N + _M * _N + i * _N + j
            first_src.append(c_idx)
            second_src.append(-1)
            dst.append(d_idx)
            for k in range(_K):
                first_src.append(i * _K + k)
                second_src.append(_M * _K + k * _N + j)
                dst.append(d_idx)
    return np.concatenate([
        np.asarray(first_src, dtype=np.int32),
        np.asarray(second_src, dtype=np.int32),
        np.asarray(dst, dtype=np.int32),
    ])


def _to_key(p):
    b = lax.bitcast_convert_type(p + 0.0, jnp.int32)
    m = lax.shift_right_arithmetic(b, 31)
    return b ^ (m & jnp.int32(0x7FFFFFFF))


def _sc_body(mem_hbm, op_hbm, tbl_hbm, out_hbm,
             op_v, skeys_iv, mem_v, tbl_v,
             st_afA, st_adA, st_afB, st_adB, st_afC, st_adC,
             part_stage, afs_v, ads_v, part_v, out_stage,
             sem_op, sem_mem, sem_tbl, sem_sc, sem_f1, sem_f2, sem_f3,
             afs_sh, ads_sh, part_sh):
    wid = lax.axis_index("s")
    has3 = (wid >= 4) & (wid < 8)

    pltpu.async_copy(op_hbm, op_v, sem_op)
    pltpu.async_copy(mem_hbm, mem_v, sem_mem)
    pltpu.async_copy(tbl_hbm, tbl_v, sem_tbl)
    pltpu.make_async_copy(op_hbm, op_v, sem_op).wait()

    def prep_body(t, carry):
        sl = pl.ds(t * _L, _L)
        skeys_iv[sl] = _to_key(lax.sort(op_v[sl]))
        return carry

    lax.fori_loop(0, _NCHUNK, prep_body, 0)

    cA = wid
    cB = wid + _NSUB
    cC = jnp.where(has3, wid + 2 * _NSUB - 4, wid)
    kAi = _to_key(op_v[pl.ds(cA * _L, _L)])
    kBi = _to_key(op_v[pl.ds(cB * _L, _L)])
    kCi = _to_key(op_v[pl.ds(cC * _L, _L)])
    zero = jnp.zeros((_L,), jnp.int32)

    def srch3(jc, carry):
        jbase = jc * _L
        out = []
        for acc, myki, c in zip(carry, (kAi, kBi, kCi), (cA, cB, cC)):
            thr = myki + jnp.where(jc < c, 1, 0)
            pos = zero + jbase
            for sz in (8, 4, 2, 1, 1):
                g = plsc.load_gather(skeys_iv, [pos + (sz - 1)])
                pos = pos + jnp.where(g < thr, sz, 0)
            out.append(acc + jnp.where(jc == c, 0, pos - jbase))
        return tuple(out)

    accA, accB, accC = lax.fori_loop(0, _NCHUNK, srch3, (zero, zero, zero))

    li = lax.iota(jnp.int32, _L)

    def diag(acc, myki):
        for m in range(_L):
            thr = myki + jnp.where(m < li, 1, 0)
            acc = acc + jnp.where(myki[m] < thr, 1, 0)
        return acc

    accA = diag(accA, kAi)
    accB = diag(accB, kBi)
    accC = diag(accC, kCi)

    pltpu.make_async_copy(mem_hbm, mem_v, sem_mem).wait()
    pltpu.make_async_copy(tbl_hbm, tbl_v, sem_tbl).wait()

    def chunk_vals(c, r, intra):
        fs = tbl_v[pl.ds(c * _L, _L)]
        ss = tbl_v[pl.ds(_NUM_OPS + c * _L, _L)]
        ds = tbl_v[pl.ds(2 * _NUM_OPS + c * _L, _L)]
        has2 = ss >= 0
        ss_c = jnp.where(has2, ss, 0)
        af = plsc.load_gather(mem_v, [fs])
        asv = plsc.load_gather(mem_v, [ss_c])
        ad = plsc.load_gather(mem_v, [ds])
        idx1 = jnp.where(r >= 1, r - 1, _NUM_OPS + li)
        hop1 = jnp.where(has2, asv - af, ad - af)
        hop2 = ad - asv
        fwd1 = jnp.maximum(hop1, 0.0)
        bwd1 = jnp.maximum(-hop1, 0.0)
        fwd2 = jnp.where(has2, jnp.maximum(hop2, 0.0), 0.0)
        bwd2 = jnp.where(has2, jnp.maximum(-hop2, 0.0), 0.0)
        intra = intra + fwd1 + bwd1 * bwd1 + fwd2 + bwd2 * bwd2
        return af, ad, idx1, intra

    afA, adA, idxA, intra = chunk_vals(cA, accA, jnp.zeros((_L,), jnp.float32))
    st_afA[...] = afA
    st_adA[...] = adA
    dA1 = pltpu.async_copy(st_afA, afs_sh.at[idxA], sem_sc)
    dA2 = pltpu.async_copy(st_adA, ads_sh.at[accA], sem_sc)

    afB, adB, idxB, intra = chunk_vals(cB, accB, intra)
    st_afB[...] = afB
    st_adB[...] = adB
    dB1 = pltpu.async_copy(st_afB, afs_sh.at[idxB], sem_sc)
    dB2 = pltpu.async_copy(st_adB, ads_sh.at[accB], sem_sc)

    part_stage[...] = intra

    @pl.when(has3)
    def _():
        afC, adC, idxC, intraC = chunk_vals(cC, accC, intra)
        st_afC[...] = afC
        st_adC[...] = adC
        pltpu.sync_copy(st_afC, afs_sh.at[idxC])
        pltpu.sync_copy(st_adC, ads_sh.at[accC])
        part_stage[...] = intraC

    pltpu.sync_copy(part_stage, part_sh.at[pl.ds(wid * _L, _L)])
    dA1.wait()
    dA2.wait()
    dB1.wait()
    dB2.wait()

    plsc.subcore_barrier()

    @pl.when(wid == 0)
    def _():
        pltpu.async_copy(afs_sh, afs_v, sem_f1)
        pltpu.async_copy(ads_sh, ads_v, sem_f2)
        pltpu.async_copy(part_sh, part_v, sem_f3)
        pltpu.make_async_copy(afs_sh, afs_v, sem_f1).wait()
        pltpu.make_async_copy(ads_sh, ads_v, sem_f2).wait()
        pltpu.make_async_copy(part_sh, part_v, sem_f3).wait()

        def inter_body(c, acc):
            a0, a1 = acc
            for cc in (c, c + _NCHUNK // 2):
                sl = pl.ds(cc * _L, _L)
                v = afs_v[sl] - ads_v[sl]
                fwd = jnp.maximum(v, 0.0)
                bwd = jnp.maximum(-v, 0.0)
                contrib = fwd + bwd * bwd
                gl = li + cc * _L
                a0, a1 = a1, a0 + jnp.where(gl < _NUM_OPS - 1, contrib, 0.0)
            return (a0, a1)

        zf = jnp.zeros((_L,), jnp.float32)
        i0, i1 = lax.fori_loop(0, _NCHUNK // 2, inter_body, (zf, zf))
        inter = i0 + i1

        def part_body(t, acc):
            a0, a1 = acc
            return (a0 + part_v[pl.ds(t * _L, _L)],
                    a1 + part_v[pl.ds((t + _NSUB // 2) * _L, _L)])

        p0, p1 = lax.fori_loop(0, _NSUB // 2, part_body, (zf, zf))
        intra_tot = p0 + p1

        inter_s = jnp.sum(inter)
        intra_s = jnp.sum(intra_tot)
        out_stage[...] = jnp.where(li == 0, inter_s,
                                   jnp.where(li == 1, intra_s, 0.0))
        pltpu.sync_copy(out_stage, out_hbm)


@jax.jit
def kernel(mem_params, op_params):
    tbl = _op_tables()
    mesh = plsc.VectorSubcoreMesh(core_axis_name="c", subcore_axis_name="s",
                                  num_cores=1)
    run = pl.kernel(
        _sc_body,
        out_type=jax.ShapeDtypeStruct((_L,), jnp.float32),
        mesh=mesh,
        compiler_params=pltpu.CompilerParams(needs_layout_passes=False),
        scratch_types=[
            pltpu.VMEM((_NUM_OPS,), jnp.float32),
            pltpu.VMEM((_NUM_OPS,), jnp.int32),
            pltpu.VMEM((_NUM_ELEMENTS,), jnp.float32),
            pltpu.VMEM((3 * _NUM_OPS,), jnp.int32),
            pltpu.VMEM((_L,), jnp.float32),
            pltpu.VMEM((_L,), jnp.float32),
            pltpu.VMEM((_L,), jnp.float32),
            pltpu.VMEM((_L,), jnp.float32),
            pltpu.VMEM((_L,), jnp.float32),
            pltpu.VMEM((_L,), jnp.float32),
            pltpu.VMEM((_L,), jnp.float32),
            pltpu.VMEM((_AFS_PAD,), jnp.float32),
            pltpu.VMEM((_NUM_OPS,), jnp.float32),
            pltpu.VMEM((_NSUB * _L,), jnp.float32),
            pltpu.VMEM((_L,), jnp.float32),
            pltpu.SemaphoreType.DMA,
            pltpu.SemaphoreType.DMA,
            pltpu.SemaphoreType.DMA,
            pltpu.SemaphoreType.DMA,
            pltpu.SemaphoreType.DMA,
            pltpu.SemaphoreType.DMA,
            pltpu.SemaphoreType.DMA,
            pltpu.VMEM_SHARED((_AFS_PAD,), jnp.float32),
            pltpu.VMEM_SHARED((_NUM_OPS,), jnp.float32),
            pltpu.VMEM_SHARED((_NSUB * _L,), jnp.float32),
        ],
    )
    out = run(mem_params, op_params, jnp.asarray(tbl))
    return (out[0], out[1])
